# baseline (speedup 1.0000x reference)
import functools

import jax
import jax.numpy as jnp
import numpy as np
from jax import lax
from jax.experimental import pallas as pl
from jax.experimental.pallas import tpu as pltpu
from jax.experimental.pallas import tpu_sc as plsc

NUM_LATENT = 32
LANES = 16
NUM_CORES = 2
NUM_SUBCORES = 16
NUM_WORKERS = NUM_CORES * NUM_SUBCORES
PACK = 128 // NUM_LATENT

_NOISE_CACHE = {}


def _rotl32(x: np.ndarray, d: int) -> np.ndarray:
    return ((x << np.uint32(d)) | (x >> np.uint32(32 - d))).astype(np.uint32)


def _threefry2x32(k0, k1, x0, x1):
    ks = [np.uint32(k0), np.uint32(k1),
          np.uint32(np.uint32(k0) ^ np.uint32(k1) ^ np.uint32(0x1BD11BDA))]
    rots = [(13, 15, 26, 6), (17, 29, 16, 24)]
    x0 = (x0 + ks[0]).astype(np.uint32)
    x1 = (x1 + ks[1]).astype(np.uint32)
    for i in range(5):
        for r in rots[i % 2]:
            x0 = (x0 + x1).astype(np.uint32)
            x1 = _rotl32(x1, r)
            x1 = (x1 ^ x0).astype(np.uint32)
        x0 = (x0 + ks[(i + 1) % 3]).astype(np.uint32)
        x1 = (x1 + ks[(i + 2) % 3] + np.uint32(i + 1)).astype(np.uint32)
    return x0, x1


def _logistic_noise(bs: int) -> np.ndarray:
    if bs not in _NOISE_CACHE:
        n = bs * NUM_LATENT
        with np.errstate(over="ignore"):
            o1, o2 = _threefry2x32(
                np.uint32(0), np.uint32(1),
                np.zeros(n, dtype=np.uint32), np.arange(n, dtype=np.uint32))
        bits = (o1 ^ o2).reshape(bs, NUM_LATENT)
        f = ((bits >> np.uint32(9)) | np.uint32(0x3F800000)).view(np.float32)
        minv = np.float32(1e-6)
        maxv = np.float32(1.0 - 1e-6)
        u = np.maximum(minv, (f - np.float32(1.0)) * (maxv - minv) + minv)
        noise = (np.log(u) - np.log(np.float32(1.0) - u)).astype(np.float32)
        b_per_w = bs // NUM_WORKERS
        noise_w = np.ascontiguousarray(
            noise.reshape(NUM_WORKERS, b_per_w, NUM_LATENT).transpose(0, 2, 1))
        _NOISE_CACHE[bs] = noise_w
    return _NOISE_CACHE[bs]


def _make_sc_kernel(bs: int, num_rows: int):
    assert bs % NUM_WORKERS == 0
    b_per_w = bs // NUM_WORKERS
    n_blocks = b_per_w // LANES
    mesh = plsc.VectorSubcoreMesh(
        core_axis_name="c", subcore_axis_name="s",
        num_cores=NUM_CORES, num_subcores=NUM_SUBCORES)

    @functools.partial(
        pl.kernel,
        mesh=mesh,
        out_type=jax.ShapeDtypeStruct((bs * NUM_LATENT // 128, 128),
                                      jnp.float32),
        scratch_types=[
            pltpu.VMEM((b_per_w,), jnp.int32),
            pltpu.VMEM((b_per_w,), jnp.int32),
            pltpu.VMEM((b_per_w, 128), jnp.float32),
            pltpu.VMEM((NUM_LATENT, b_per_w), jnp.float32),
            pltpu.VMEM((b_per_w * NUM_LATENT // 128, 128), jnp.float32),
            pltpu.SemaphoreType.DMA,
        ],
        compiler_params=pltpu.CompilerParams(needs_layout_passes=False),
    )
    def gumbel_gather(table_hbm, idx_hbm, noise_hbm, out_hbm,
                      idx_v, idx4_v, rows_v, noise_v, out_v, sem):
        wid = lax.axis_index("s") * NUM_CORES + lax.axis_index("c")
        base = wid * b_per_w
        pltpu.sync_copy(idx_hbm.at[pl.ds(base, b_per_w)], idx_v)

        def shift_body(i, carry):
            a = idx_v[pl.ds(i * LANES, LANES)]
            idx4_v[pl.ds(i * LANES, LANES)] = lax.shift_right_logical(a, 2)
            return carry

        lax.fori_loop(0, b_per_w // LANES, shift_body, 0, unroll=4)

        gather = pltpu.async_copy(table_hbm.at[idx4_v], rows_v, sem)
        pltpu.sync_copy(noise_hbm.at[wid], noise_v)
        gather.wait()

        lane_iota = lax.iota(jnp.int32, LANES)

        def block_body(blk, carry):
            b0 = blk * LANES
            a16 = idx_v[pl.ds(b0, LANES)]
            col0 = lax.shift_left(
                jnp.bitwise_and(a16, jnp.int32(PACK - 1)),
                jnp.int32(5))
            row16 = lane_iota + b0
            flat0 = lax.shift_left(row16, jnp.int32(5))
            for j in range(NUM_LATENT):
                g = plsc.load_gather(rows_v, [row16, col0 + jnp.int32(j)])
                t = noise_v[j, pl.ds(b0, LANES)]
                y = jnp.where(g + t > 0.0, 1.0, 0.0).astype(jnp.float32)
                flat = flat0 + jnp.int32(j)
                plsc.store_scatter(
                    out_v,
                    [lax.shift_right_logical(flat, 7),
                     jnp.bitwise_and(flat, jnp.int32(127))],
                    y)
            return carry

        lax.fori_loop(0, n_blocks, block_body, 0)
        rows_out = b_per_w * NUM_LATENT // 128
        pltpu.sync_copy(out_v, out_hbm.at[pl.ds(wid * rows_out, rows_out)])

    return gumbel_gather


def kernel(action, log_alpha):
    bs = action.shape[0]
    num_action = log_alpha.shape[0]
    assert num_action % PACK == 0
    table128 = jnp.reshape(log_alpha, (num_action // PACK, 128))
    noise = jnp.asarray(_logistic_noise(bs))
    sc = _make_sc_kernel(bs, num_action)
    packed = sc(table128, action, noise)
    return jnp.reshape(packed, (bs, NUM_LATENT))


# speedup vs baseline: 3.9373x; 3.9373x over previous
import functools

import jax
import jax.numpy as jnp
import numpy as np
from jax import lax
from jax.experimental import pallas as pl
from jax.experimental.pallas import tpu as pltpu
from jax.experimental.pallas import tpu_sc as plsc

NUM_LATENT = 32
LANES = 16
NUM_CORES = 2
NUM_SUBCORES = 16
NUM_WORKERS = NUM_CORES * NUM_SUBCORES
NJT = NUM_LATENT // 8

_NOISE_CACHE = {}


def _rotl32(x: np.ndarray, d: int) -> np.ndarray:
    return ((x << np.uint32(d)) | (x >> np.uint32(32 - d))).astype(np.uint32)


def _threefry2x32(k0, k1, x0, x1):
    ks = [np.uint32(k0), np.uint32(k1),
          np.uint32(np.uint32(k0) ^ np.uint32(k1) ^ np.uint32(0x1BD11BDA))]
    rots = [(13, 15, 26, 6), (17, 29, 16, 24)]
    x0 = (x0 + ks[0]).astype(np.uint32)
    x1 = (x1 + ks[1]).astype(np.uint32)
    for i in range(5):
        for r in rots[i % 2]:
            x0 = (x0 + x1).astype(np.uint32)
            x1 = _rotl32(x1, r)
            x1 = (x1 ^ x0).astype(np.uint32)
        x0 = (x0 + ks[(i + 1) % 3]).astype(np.uint32)
        x1 = (x1 + ks[(i + 2) % 3] + np.uint32(i + 1)).astype(np.uint32)
    return x0, x1


def _logistic_noise_wjr(bs: int) -> np.ndarray:
    if bs not in _NOISE_CACHE:
        n = bs * NUM_LATENT
        with np.errstate(over="ignore"):
            o1, o2 = _threefry2x32(
                np.uint32(0), np.uint32(1),
                np.zeros(n, dtype=np.uint32), np.arange(n, dtype=np.uint32))
        bits = (o1 ^ o2).reshape(bs, NUM_LATENT)
        f = ((bits >> np.uint32(9)) | np.uint32(0x3F800000)).view(np.float32)
        minv = np.float32(1e-6)
        maxv = np.float32(1.0 - 1e-6)
        u = np.maximum(minv, (f - np.float32(1.0)) * (maxv - minv) + minv)
        noise = (np.log(u) - np.log(np.float32(1.0) - u)).astype(np.float32)
        b_per_w = bs // NUM_WORKERS
        noise_w = np.ascontiguousarray(
            noise.reshape(NUM_WORKERS, b_per_w, NUM_LATENT)
            .transpose(0, 2, 1)
            .reshape(NUM_WORKERS, NJT, 8, b_per_w // 128, 128)
            .transpose(0, 1, 3, 2, 4))
        _NOISE_CACHE[bs] = noise_w
    return _NOISE_CACHE[bs]


def _make_sc_kernel(bs: int):
    assert bs % (NUM_WORKERS * LANES) == 0
    b_per_w = bs // NUM_WORKERS
    n_blocks = b_per_w // LANES
    mesh = plsc.VectorSubcoreMesh(
        core_axis_name="c", subcore_axis_name="s",
        num_cores=NUM_CORES, num_subcores=NUM_SUBCORES)

    @functools.partial(
        pl.kernel,
        mesh=mesh,
        out_type=jax.ShapeDtypeStruct((NJT, 8, bs), jnp.float32),
        scratch_types=[
            pltpu.VMEM((b_per_w,), jnp.int32),
            pltpu.VMEM((LANES, NJT, 8, 128), jnp.float32),
            pltpu.VMEM((NJT, b_per_w // 128, 8, 128), jnp.float32),
            pltpu.VMEM((NJT, b_per_w // 128, 8, 128), jnp.float32),
            pltpu.SemaphoreType.DMA,
            pltpu.SemaphoreType.DMA,
        ],
        compiler_params=pltpu.CompilerParams(needs_layout_passes=False),
    )
    def gumbel_gather(table_hbm, idx_hbm, noise_hbm, out_hbm,
                      idx_v, stage_v, cols_v, noise_v, sem0, semn):
        wid = lax.axis_index("s") * NUM_CORES + lax.axis_index("c")
        base = wid * b_per_w
        pltpu.sync_copy(idx_hbm.at[pl.ds(base, b_per_w)], idx_v)
        noise_cp = pltpu.async_copy(noise_hbm.at[wid], noise_v, semn)
        noise_cp.wait()

        lane_iota = lax.iota(jnp.int32, LANES)

        def block_body(blk, carry):
            b0 = blk * LANES
            a16 = idx_v[pl.ds(b0, LANES)]
            a_tile = jnp.bitwise_and(a16, jnp.int32(-128))
            copies = []
            for i in range(LANES):
                t0 = pl.multiple_of(a_tile[i], 128)
                for J in range(NJT):
                    copies.append(pltpu.async_copy(
                        table_hbm.at[pl.ds(8 * J, 8), pl.ds(t0, 128)],
                        stage_v.at[i, J],
                        sem0))
            for cp in copies:
                cp.wait()
            lane_sel = jnp.bitwise_and(a16, jnp.int32(127))
            c_out = blk // 8
            l_out = (blk % 8) * LANES
            for J in range(NJT):
                j_v = jnp.full((LANES,), J, jnp.int32)
                for js in range(8):
                    g = plsc.load_gather(
                        stage_v,
                        [lane_iota, j_v,
                         jnp.full((LANES,), js, jnp.int32), lane_sel])
                    t = noise_v[J, c_out, js, pl.ds(l_out, LANES)]
                    cols_v[J, c_out, js, pl.ds(l_out, LANES)] = jnp.where(
                        g + t > 0.0, 1.0, 0.0).astype(jnp.float32)
            return carry

        lax.fori_loop(0, n_blocks, block_body, 0)

        for J in range(NJT):
            for c in range(b_per_w // 128):
                pltpu.sync_copy(
                    cols_v.at[J, c],
                    out_hbm.at[J, :, pl.ds(base + 128 * c, 128)])

    return gumbel_gather


def kernel(action, log_alpha):
    bs = action.shape[0]
    table_t = log_alpha.T
    noise = jnp.asarray(_logistic_noise_wjr(bs))
    sc = _make_sc_kernel(bs)
    packed = sc(table_t, action, noise)
    return jnp.reshape(packed, (NUM_LATENT, bs)).T
